# baseline (speedup 1.0000x reference)
import functools

import jax
import jax.numpy as jnp
from jax.experimental import pallas as pl
from jax.experimental.pallas import tpu as pltpu

_C = 512
_B = 8
_L = 1024
_NTOK = _B * _L
_STEPS = 3
_CHUNK = 512
_NCHUNK = _NTOK // _CHUNK
_GCHUNK = 512
_PREC_MM = jax.lax.Precision.DEFAULT
_NEG = -1e30
_F32 = jnp.float32
_BF16 = jnp.bfloat16


def _lstm_act(gates, c):
    ig = jax.nn.sigmoid(gates[:, 0 * _C:1 * _C])
    fg = jax.nn.sigmoid(gates[:, 1 * _C:2 * _C])
    gg = jnp.tanh(gates[:, 2 * _C:3 * _C])
    og = jax.nn.sigmoid(gates[:, 3 * _C:4 * _C])
    c = fg * c + ig * gg
    return og * jnp.tanh(c), c


def _split(v):
    hi = v.astype(_BF16)
    lo = (v - hi.astype(_F32)).astype(_BF16)
    return hi, lo


def _mm(a, b):
    return jax.lax.dot_general(a, b, (((1,), (0,)), ((), ())),
                               precision=_PREC_MM,
                               preferred_element_type=_F32)


def _split_mm(v, bhi_c, blo_c):
    hi, lo = _split(v)
    ab = _mm(jnp.concatenate([hi, lo], axis=0), bhi_c)
    return ab[:_B] + ab[_B:] + _mm(hi, blo_c)


def _fused_body(x1t_ref, x2t_ref, x1r_ref, x2r_ref, batch_ref, qstar_ref,
                w_ref, wihT_ref, whhT_ref, wcT_ref, wrT_ref, b_ref,
                out_ref, h_ref, g_ref):
    for i in range(_B):
        wrow = w_ref[i:i + 1, :]
        segx = x1r_ref[pl.ds(i * _L, _L), :]
        h_ref[i:i + 1, :] = _mm(wrow, segx)
    h = h_ref[...]
    c = jnp.zeros((_B, _C), _F32)
    bias = b_ref[...]
    iota_b = jax.lax.broadcasted_iota(jnp.int32, (_B, _CHUNK), 0)

    def gates_of(lhs1, w1T_ref, lhs2, w2T_ref):
        def gate_chunk(g, _):
            w1c = w1T_ref[:, pl.ds(g * _GCHUNK, _GCHUNK)]
            w2c = w2T_ref[:, pl.ds(g * _GCHUNK, _GCHUNK)]
            g_ref[:, pl.ds(g * _GCHUNK, _GCHUNK)] = (
                jax.lax.dot_general(lhs1, w1c, (((1,), (0,)), ((), ())),
                                    precision=_PREC_MM)
                + jax.lax.dot_general(lhs2, w2c, (((1,), (0,)), ((), ())),
                                      precision=_PREC_MM))
            return 0

        jax.lax.fori_loop(0, (4 * _C) // _GCHUNK, gate_chunk, 0)
        return g_ref[...]

    def pool(h):
        def chunk(j, carry):
            m, denom, racc = carry
            x1tc = x1t_ref[:, pl.ds(j * _CHUNK, _CHUNK)]
            x2tc = x2t_ref[:, pl.ds(j * _CHUNK, _CHUNK)]
            segc = batch_ref[:, pl.ds(j * _CHUNK, _CHUNK)]
            oh = iota_b == segc
            s = _split_mm(h, x1tc, x2tc)
            smask = jnp.where(oh, s, _NEG)
            m_new = jnp.maximum(m, jnp.max(smask, axis=1, keepdims=True))
            scale = jnp.exp(m - m_new)
            p = jnp.exp(smask - m_new)
            denom = denom * scale + jnp.sum(p, axis=1, keepdims=True)
            x1rc = x1r_ref[pl.ds(j * _CHUNK, _CHUNK), :]
            x2rc = x2r_ref[pl.ds(j * _CHUNK, _CHUNK), :]
            racc = racc * scale + _split_mm(p, x1rc, x2rc)
            return m_new, denom, racc

        m0 = jnp.full((_B, 1), _NEG, _F32)
        z1 = jnp.zeros((_B, 1), _F32)
        z2 = jnp.zeros((_B, _C), _F32)
        _, denom, racc = jax.lax.fori_loop(0, _NCHUNK, chunk, (m0, z1, z2))
        return racc / (denom + 1e-16)

    qs = qstar_ref[...]
    h, c = _lstm_act(gates_of(qs, wihT_ref, h, whhT_ref) + bias, c)
    r = pool(h)

    for _ in range(_STEPS - 1):
        h, c = _lstm_act(gates_of(h, wcT_ref, r, wrT_ref) + bias, c)
        r = pool(h)

    out_ref[...] = jnp.concatenate([h, r], axis=1)


@functools.partial(jax.jit, static_argnames=("interpret",))
def _run_fused(x1t, x2t, x1r, x2r, seg_row, q_star, w_rows,
               WihT, WhhT, WcT, WrT, bias, interpret=False):
    return pl.pallas_call(
        _fused_body,
        out_shape=jax.ShapeDtypeStruct((_B, 2 * _C), _F32),
        scratch_shapes=[
            pltpu.VMEM((_B, _C), _F32),
            pltpu.VMEM((_B, 4 * _C), _F32),
        ],
        interpret=interpret,
    )(x1t, x2t, x1r, x2r, seg_row, q_star, w_rows, WihT, WhhT, WcT, WrT,
      bias)


def kernel(x, batch, q_star, bank_s_list, bank_s, index, cuda,
           W_ih, W_hh, b_ih, b_hh, interpret=False):
    w_rows = jax.lax.dynamic_slice_in_dim(
        bank_s_list, index, 1, axis=1).reshape(_B, _L).astype(_BF16)
    seg_row = batch.astype(jnp.int32).reshape(1, _NTOK)
    bias = jnp.broadcast_to((b_ih + b_hh).reshape(1, 4 * _C), (_B, 4 * _C))
    x1r = x.astype(_BF16)
    x2r = (x - x1r.astype(_F32)).astype(_BF16)
    x1t = x1r.T
    x2t = x2r.T
    WihT = W_ih.T
    WhhT = W_hh.T
    WcT = WihT[:_C, :] + WhhT
    WrT = WihT[_C:, :]
    return _run_fused(x1t, x2t, x1r, x2r, seg_row, q_star, w_rows,
                      WihT, WhhT, WcT, WrT, bias, interpret=interpret)


# speedup vs baseline: 2.0409x; 2.0409x over previous
import functools

import jax
import jax.numpy as jnp
from jax.experimental import pallas as pl
from jax.experimental.pallas import tpu as pltpu

_C = 512
_B = 8
_L = 1024
_NTOK = _B * _L
_STEPS = 3
_CHUNK = 512
_NCHUNK = _NTOK // _CHUNK
_GCHUNK = 512
_PREC_MM = jax.lax.Precision.DEFAULT
_NEG = -1e30
_F32 = jnp.float32
_BF16 = jnp.bfloat16


def _lstm_act(gates, c):
    ig = jax.nn.sigmoid(gates[:, 0 * _C:1 * _C])
    fg = jax.nn.sigmoid(gates[:, 1 * _C:2 * _C])
    gg = jnp.tanh(gates[:, 2 * _C:3 * _C])
    og = jax.nn.sigmoid(gates[:, 3 * _C:4 * _C])
    c = fg * c + ig * gg
    return og * jnp.tanh(c), c


def _split(v):
    hi = v.astype(_BF16)
    lo = (v - hi.astype(_F32)).astype(_BF16)
    return hi, lo


def _mm(a, b):
    return jax.lax.dot_general(a, b, (((1,), (0,)), ((), ())),
                               precision=_PREC_MM,
                               preferred_element_type=_F32)


def _split_mm(v, bhi, blo):
    hi, lo = _split(v)
    ab = _mm(jnp.concatenate([hi, lo], axis=0), bhi)
    return ab[:_B] + ab[_B:] + _mm(hi, blo)


def _fused_body(xc_ref, batch_ref, qstar_ref, w_ref, wih_ref, whh_ref,
                b_ref, eye_ref, out_ref,
                x1r_ref, x2r_ref, x1t_ref, x2t_ref, wc_ref, h_ref, g_ref):
    i = pl.program_id(0)

    @pl.when(i < _NCHUNK)
    def _build():
        xc = xc_ref[...]
        hi, lo = _split(xc)
        x1r_ref[pl.ds(i * _CHUNK, _CHUNK), :] = hi
        x2r_ref[pl.ds(i * _CHUNK, _CHUNK), :] = lo
        eye = eye_ref[...]
        th = jax.lax.dot_general(eye, hi, (((1,), (1,)), ((), ())),
                                 precision=_PREC_MM,
                                 preferred_element_type=_F32)
        x1t_ref[:, pl.ds(i * _CHUNK, _CHUNK)] = th.astype(_BF16)
        tl = jax.lax.dot_general(eye, lo, (((1,), (1,)), ((), ())),
                                 precision=_PREC_MM,
                                 preferred_element_type=_F32)
        x2t_ref[:, pl.ds(i * _CHUNK, _CHUNK)] = tl.astype(_BF16)

    @pl.when(i < _NCHUNK)
    def _asit():
        xc = xc_ref[...]
        seg = i // 2
        half = i % 2
        wrow = w_ref[pl.ds(seg, 1), pl.ds(half * _CHUNK, _CHUNK)]
        part = jax.lax.dot_general(wrow, xc, (((1,), (0,)), ((), ())),
                                   precision=_PREC_MM)
        @pl.when(half == 0)
        def _():
            h_ref[pl.ds(seg, 1), :] = part

        @pl.when(half == 1)
        def _():
            h_ref[pl.ds(seg, 1), :] = h_ref[pl.ds(seg, 1), :] + part

    @pl.when(i == _NCHUNK)
    def _main():
        def wc_chunk(g, _):
            wc_ref[pl.ds(g * _GCHUNK, _GCHUNK), :] = (
                wih_ref[pl.ds(g * _GCHUNK, _GCHUNK), 0:_C]
                + whh_ref[pl.ds(g * _GCHUNK, _GCHUNK), :])
            return 0

        jax.lax.fori_loop(0, (4 * _C) // _GCHUNK, wc_chunk, 0)

        h = h_ref[...]
        c = jnp.zeros((_B, _C), _F32)
        bias = b_ref[...]
        iota_b = jax.lax.broadcasted_iota(jnp.int32, (_B, _CHUNK), 0)

        def gates_2(lhs1, lhs2, split_w1):
            def gate_chunk(g, _):
                gsl = pl.ds(g * _GCHUNK, _GCHUNK)
                if split_w1:
                    w1c = wih_ref[gsl, :]
                    w2c = whh_ref[gsl, :]
                else:
                    w1c = wc_ref[gsl, :]
                    w2c = wih_ref[gsl, _C:2 * _C]
                g_ref[:, gsl] = (
                    jax.lax.dot_general(lhs1, w1c, (((1,), (1,)), ((), ())),
                                        precision=_PREC_MM)
                    + jax.lax.dot_general(lhs2, w2c, (((1,), (1,)), ((), ())),
                                          precision=_PREC_MM))
                return 0

            jax.lax.fori_loop(0, (4 * _C) // _GCHUNK, gate_chunk, 0)
            return g_ref[...]

        def pool(h):
            def chunk(j, carry):
                m, denom, racc = carry
                tsl = pl.ds(j * _CHUNK, _CHUNK)
                segc = batch_ref[:, tsl]
                oh = iota_b == segc
                s = _split_mm(h, x1t_ref[:, tsl], x2t_ref[:, tsl])
                smask = jnp.where(oh, s, _NEG)
                m_new = jnp.maximum(m, jnp.max(smask, axis=1, keepdims=True))
                scale = jnp.exp(m - m_new)
                p = jnp.exp(smask - m_new)
                denom = denom * scale + jnp.sum(p, axis=1, keepdims=True)
                racc = racc * scale + _split_mm(
                    p, x1r_ref[tsl, :], x2r_ref[tsl, :])
                return m_new, denom, racc

            m0 = jnp.full((_B, 1), _NEG, _F32)
            z1 = jnp.zeros((_B, 1), _F32)
            z2 = jnp.zeros((_B, _C), _F32)
            _, denom, racc = jax.lax.fori_loop(0, _NCHUNK, chunk,
                                               (m0, z1, z2))
            return racc / (denom + 1e-16)

        qs = qstar_ref[...]
        h, c = _lstm_act(gates_2(qs, h, True) + bias, c)
        r = pool(h)
        for _ in range(_STEPS - 1):
            h, c = _lstm_act(gates_2(h, r, False) + bias, c)
            r = pool(h)

        out_ref[...] = jnp.concatenate([h, r], axis=1)


@functools.partial(jax.jit, static_argnames=("interpret",))
def _run_fused(x, seg_row, q_star, w_rows, W_ih, W_hh, bias, eye,
               interpret=False):
    grid = (_NCHUNK + 1,)
    return pl.pallas_call(
        _fused_body,
        grid=grid,
        in_specs=[
            pl.BlockSpec((_CHUNK, _C),
                         lambda i: (jnp.minimum(i, _NCHUNK - 1), 0)),
            pl.BlockSpec((1, _NTOK), lambda i: (0, 0)),
            pl.BlockSpec((_B, 2 * _C), lambda i: (0, 0)),
            pl.BlockSpec((_B, _L), lambda i: (0, 0)),
            pl.BlockSpec((4 * _C, 2 * _C), lambda i: (0, 0)),
            pl.BlockSpec((4 * _C, _C), lambda i: (0, 0)),
            pl.BlockSpec((_B, 4 * _C), lambda i: (0, 0)),
            pl.BlockSpec((_C, _C), lambda i: (0, 0)),
        ],
        out_specs=pl.BlockSpec((_B, 2 * _C), lambda i: (0, 0)),
        out_shape=jax.ShapeDtypeStruct((_B, 2 * _C), _F32),
        scratch_shapes=[
            pltpu.VMEM((_NTOK, _C), _BF16),
            pltpu.VMEM((_NTOK, _C), _BF16),
            pltpu.VMEM((_C, _NTOK), _BF16),
            pltpu.VMEM((_C, _NTOK), _BF16),
            pltpu.VMEM((4 * _C, _C), _F32),
            pltpu.VMEM((_B, _C), _F32),
            pltpu.VMEM((_B, 4 * _C), _F32),
        ],
        interpret=interpret,
    )(x, seg_row, q_star, w_rows, W_ih, W_hh, bias, eye)


def kernel(x, batch, q_star, bank_s_list, bank_s, index, cuda,
           W_ih, W_hh, b_ih, b_hh, interpret=False):
    w_rows = jax.lax.dynamic_slice_in_dim(
        bank_s_list, index, 1, axis=1).reshape(_B, _L)
    seg_row = batch.astype(jnp.int32).reshape(1, _NTOK)
    bias = jnp.broadcast_to((b_ih + b_hh).reshape(1, 4 * _C), (_B, 4 * _C))
    eye = jnp.eye(_C, dtype=_BF16)
    return _run_fused(x, seg_row, q_star, w_rows, W_ih, W_hh, bias, eye,
                      interpret=interpret)


# speedup vs baseline: 2.3185x; 1.1360x over previous
import functools

import jax
import jax.numpy as jnp
from jax.experimental import pallas as pl
from jax.experimental.pallas import tpu as pltpu

_C = 512
_B = 8
_L = 1024
_NTOK = _B * _L
_STEPS = 3
_CHUNK = 512
_NCHUNK = _NTOK // _CHUNK
_GCHUNK = 512
_PREC_MM = jax.lax.Precision.DEFAULT
_NEG = -1e30
_F32 = jnp.float32
_BF16 = jnp.bfloat16


def _lstm_act(gates, c):
    ig = jax.nn.sigmoid(gates[:, 0 * _C:1 * _C])
    fg = jax.nn.sigmoid(gates[:, 1 * _C:2 * _C])
    gg = jnp.tanh(gates[:, 2 * _C:3 * _C])
    og = jax.nn.sigmoid(gates[:, 3 * _C:4 * _C])
    c = fg * c + ig * gg
    return og * jnp.tanh(c), c


def _split(v):
    hi = v.astype(_BF16)
    lo = (v - hi.astype(_F32)).astype(_BF16)
    return hi, lo


def _mm(a, b):
    return jax.lax.dot_general(a, b, (((1,), (0,)), ((), ())),
                               precision=_PREC_MM,
                               preferred_element_type=_F32)


def _split_mm(v, bhi, blo):
    hi, lo = _split(v)
    ab = _mm(jnp.concatenate([hi, lo], axis=0), bhi)
    return ab[:_B] + ab[_B:] + _mm(hi, blo)


def _fused_body(xc_ref, batch_ref, qstar_ref, w_ref, wih_ref, whh_ref,
                b_ref, eye_ref, out_ref,
                x1r_ref, x2r_ref, x1t_ref, x2t_ref, wc_ref, h_ref, g_ref):
    i = pl.program_id(0)

    @pl.when(i < _NCHUNK)
    def _build():
        xc = xc_ref[...]
        hi, lo = _split(xc)
        x1r_ref[pl.ds(i * _CHUNK, _CHUNK), :] = hi
        x2r_ref[pl.ds(i * _CHUNK, _CHUNK), :] = lo
        eye = eye_ref[...]
        th = jax.lax.dot_general(eye, hi, (((1,), (1,)), ((), ())),
                                 precision=_PREC_MM,
                                 preferred_element_type=_F32)
        x1t_ref[:, pl.ds(i * _CHUNK, _CHUNK)] = th.astype(_BF16)
        tl = jax.lax.dot_general(eye, lo, (((1,), (1,)), ((), ())),
                                 precision=_PREC_MM,
                                 preferred_element_type=_F32)
        x2t_ref[:, pl.ds(i * _CHUNK, _CHUNK)] = tl.astype(_BF16)

    @pl.when(i < _NCHUNK)
    def _asit():
        xc = xc_ref[...]
        seg = i // 2
        half = i % 2
        wrow = w_ref[pl.ds(seg, 1), pl.ds(half * _CHUNK, _CHUNK)]
        part = jax.lax.dot_general(wrow, xc, (((1,), (0,)), ((), ())),
                                   precision=_PREC_MM)
        @pl.when(half == 0)
        def _():
            h_ref[pl.ds(seg, 1), :] = part

        @pl.when(half == 1)
        def _():
            h_ref[pl.ds(seg, 1), :] = h_ref[pl.ds(seg, 1), :] + part

    @pl.when(i == _NCHUNK)
    def _main():
        def wc_chunk(g, _):
            wc_ref[pl.ds(g * _GCHUNK, _GCHUNK), :] = (
                wih_ref[pl.ds(g * _GCHUNK, _GCHUNK), 0:_C]
                + whh_ref[pl.ds(g * _GCHUNK, _GCHUNK), :])
            return 0

        jax.lax.fori_loop(0, (4 * _C) // _GCHUNK, wc_chunk, 0)

        h = h_ref[...]
        c = jnp.zeros((_B, _C), _F32)
        bias = b_ref[...]
        iota_b = jax.lax.broadcasted_iota(jnp.int32, (_B, _CHUNK), 0)

        def gates_2(lhs1, lhs2, split_w1):
            def gate_chunk(g, _):
                gsl = pl.ds(g * _GCHUNK, _GCHUNK)
                if split_w1:
                    w1c = wih_ref[gsl, :]
                    w2c = whh_ref[gsl, :]
                else:
                    w1c = wc_ref[gsl, :]
                    w2c = wih_ref[gsl, _C:2 * _C]
                g_ref[:, gsl] = (
                    jax.lax.dot_general(lhs1, w1c, (((1,), (1,)), ((), ())),
                                        precision=_PREC_MM)
                    + jax.lax.dot_general(lhs2, w2c, (((1,), (1,)), ((), ())),
                                          precision=_PREC_MM))
                return 0

            jax.lax.fori_loop(0, (4 * _C) // _GCHUNK, gate_chunk, 0)
            return g_ref[...]

        def pool(h):
            h1, h2 = _split(h)
            hh = jnp.concatenate([h1, h2], axis=0)

            def chunk4(j4, carry):
                m, denom, racc = carry
                for u in range(4):
                    j = j4 * 4 + u
                    tsl = pl.ds(j * _CHUNK, _CHUNK)
                    segc = batch_ref[:, tsl]
                    oh = iota_b == segc
                    sab = _mm(hh, x1t_ref[:, tsl])
                    s = sab[:_B] + sab[_B:] + _mm(h1, x2t_ref[:, tsl])
                    smask = jnp.where(oh, s, _NEG)
                    m_new = jnp.maximum(
                        m, jnp.max(smask, axis=1, keepdims=True))
                    scale = jnp.exp(m - m_new)
                    p = jnp.exp(smask - m_new)
                    denom = (denom * scale
                             + jnp.sum(p, axis=1, keepdims=True))
                    p1, p2 = _split(p)
                    pp = jnp.concatenate([p1, p2], axis=0)
                    rab = _mm(pp, x1r_ref[tsl, :])
                    racc = racc * scale + (rab[:_B] + rab[_B:])
                    m = m_new
                return m, denom, racc

            m0 = jnp.full((_B, 1), _NEG, _F32)
            z1 = jnp.zeros((_B, 1), _F32)
            z2 = jnp.zeros((_B, _C), _F32)
            _, denom, racc = jax.lax.fori_loop(0, _NCHUNK // 4, chunk4,
                                               (m0, z1, z2))
            return racc / (denom + 1e-16)

        qs = qstar_ref[...]
        h, c = _lstm_act(gates_2(qs, h, True) + bias, c)
        r = pool(h)
        for _ in range(_STEPS - 1):
            h, c = _lstm_act(gates_2(h, r, False) + bias, c)
            r = pool(h)

        out_ref[...] = jnp.concatenate([h, r], axis=1)


@functools.partial(jax.jit, static_argnames=("interpret",))
def _run_fused(x, seg_row, q_star, w_rows, W_ih, W_hh, bias, eye,
               interpret=False):
    grid = (_NCHUNK + 1,)
    return pl.pallas_call(
        _fused_body,
        grid=grid,
        in_specs=[
            pl.BlockSpec((_CHUNK, _C),
                         lambda i: (jnp.minimum(i, _NCHUNK - 1), 0)),
            pl.BlockSpec((1, _NTOK), lambda i: (0, 0)),
            pl.BlockSpec((_B, 2 * _C), lambda i: (0, 0)),
            pl.BlockSpec((_B, _L), lambda i: (0, 0)),
            pl.BlockSpec((4 * _C, 2 * _C), lambda i: (0, 0)),
            pl.BlockSpec((4 * _C, _C), lambda i: (0, 0)),
            pl.BlockSpec((_B, 4 * _C), lambda i: (0, 0)),
            pl.BlockSpec((_C, _C), lambda i: (0, 0)),
        ],
        out_specs=pl.BlockSpec((_B, 2 * _C), lambda i: (0, 0)),
        out_shape=jax.ShapeDtypeStruct((_B, 2 * _C), _F32),
        scratch_shapes=[
            pltpu.VMEM((_NTOK, _C), _BF16),
            pltpu.VMEM((_NTOK, _C), _BF16),
            pltpu.VMEM((_C, _NTOK), _BF16),
            pltpu.VMEM((_C, _NTOK), _BF16),
            pltpu.VMEM((4 * _C, _C), _F32),
            pltpu.VMEM((_B, _C), _F32),
            pltpu.VMEM((_B, 4 * _C), _F32),
        ],
        interpret=interpret,
    )(x, seg_row, q_star, w_rows, W_ih, W_hh, bias, eye)


def kernel(x, batch, q_star, bank_s_list, bank_s, index, cuda,
           W_ih, W_hh, b_ih, b_hh, interpret=False):
    w_rows = jax.lax.dynamic_slice_in_dim(
        bank_s_list, index, 1, axis=1).reshape(_B, _L)
    seg_row = batch.astype(jnp.int32).reshape(1, _NTOK)
    bias = jnp.broadcast_to((b_ih + b_hh).reshape(1, 4 * _C), (_B, 4 * _C))
    eye = jnp.eye(_C, dtype=_BF16)
    return _run_fused(x, seg_row, q_star, w_rows, W_ih, W_hh, bias, eye,
                      interpret=interpret)


# speedup vs baseline: 2.4359x; 1.0507x over previous
import functools

import jax
import jax.numpy as jnp
from jax.experimental import pallas as pl
from jax.experimental.pallas import tpu as pltpu

_C = 512
_B = 8
_L = 1024
_NTOK = _B * _L
_STEPS = 3
_CHUNK = 512
_NCHUNK = _NTOK // _CHUNK
_GCHUNK = 512
_PREC_MM = jax.lax.Precision.DEFAULT
_NEG = -1e30
_F32 = jnp.float32
_BF16 = jnp.bfloat16


def _lstm_act(gates, c):
    ig = jax.nn.sigmoid(gates[:, 0 * _C:1 * _C])
    fg = jax.nn.sigmoid(gates[:, 1 * _C:2 * _C])
    gg = jnp.tanh(gates[:, 2 * _C:3 * _C])
    og = jax.nn.sigmoid(gates[:, 3 * _C:4 * _C])
    c = fg * c + ig * gg
    return og * jnp.tanh(c), c


def _split(v):
    hi = v.astype(_BF16)
    lo = (v - hi.astype(_F32)).astype(_BF16)
    return hi, lo


def _mm(a, b):
    return jax.lax.dot_general(a, b, (((1,), (0,)), ((), ())),
                               precision=_PREC_MM,
                               preferred_element_type=_F32)


def _split_mm(v, bhi, blo):
    hi, lo = _split(v)
    ab = _mm(jnp.concatenate([hi, lo], axis=0), bhi)
    return ab[:_B] + ab[_B:] + _mm(hi, blo)


def _fused_body(xc_ref, batch_ref, qstar_ref, w_ref, wih_ref, whh_ref,
                b_ref, eye_ref, out_ref,
                x1r_ref, x2r_ref, x1t_ref, x2t_ref, wc_ref, h_ref, g_ref):
    i = pl.program_id(0)

    @pl.when(i < _NCHUNK)
    def _build():
        xc = xc_ref[...]
        hi, lo = _split(xc)
        x1r_ref[pl.ds(i * _CHUNK, _CHUNK), :] = hi
        x2r_ref[pl.ds(i * _CHUNK, _CHUNK), :] = lo
        eye = eye_ref[...]
        th = jax.lax.dot_general(eye, hi, (((1,), (1,)), ((), ())),
                                 precision=_PREC_MM,
                                 preferred_element_type=_F32)
        x1t_ref[:, pl.ds(i * _CHUNK, _CHUNK)] = th.astype(_BF16)
        tl = jax.lax.dot_general(eye, lo, (((1,), (1,)), ((), ())),
                                 precision=_PREC_MM,
                                 preferred_element_type=_F32)
        x2t_ref[:, pl.ds(i * _CHUNK, _CHUNK)] = tl.astype(_BF16)

    @pl.when(i < _NCHUNK)
    def _asit():
        xc = xc_ref[...]
        seg = i // 2
        half = i % 2
        wrow = w_ref[pl.ds(seg, 1), pl.ds(half * _CHUNK, _CHUNK)]
        part = jax.lax.dot_general(wrow, xc, (((1,), (0,)), ((), ())),
                                   precision=_PREC_MM)
        @pl.when(half == 0)
        def _():
            h_ref[pl.ds(seg, 1), :] = part

        @pl.when(half == 1)
        def _():
            h_ref[pl.ds(seg, 1), :] = h_ref[pl.ds(seg, 1), :] + part

    @pl.when(i == _NCHUNK)
    def _main():
        def wc_chunk(g, _):
            wc_ref[pl.ds(g * _GCHUNK, _GCHUNK), :] = (
                wih_ref[pl.ds(g * _GCHUNK, _GCHUNK), 0:_C]
                + whh_ref[pl.ds(g * _GCHUNK, _GCHUNK), :])
            return 0

        jax.lax.fori_loop(0, (4 * _C) // _GCHUNK, wc_chunk, 0)

        h = h_ref[...]
        c = jnp.zeros((_B, _C), _F32)
        bias = b_ref[...]
        iota_b = jax.lax.broadcasted_iota(jnp.int32, (_B, _CHUNK), 0)

        def gates_2(lhs1, lhs2, split_w1):
            def gate_chunk(g, _):
                gsl = pl.ds(g * _GCHUNK, _GCHUNK)
                if split_w1:
                    w1c = wih_ref[gsl, :]
                    w2c = whh_ref[gsl, :]
                else:
                    w1c = wc_ref[gsl, :]
                    w2c = wih_ref[gsl, _C:2 * _C]
                g_ref[:, gsl] = (
                    jax.lax.dot_general(lhs1, w1c, (((1,), (1,)), ((), ())),
                                        precision=_PREC_MM)
                    + jax.lax.dot_general(lhs2, w2c, (((1,), (1,)), ((), ())),
                                          precision=_PREC_MM))
                return 0

            for g in range((4 * _C) // _GCHUNK):
                gate_chunk(g, 0)
            return g_ref[...]

        def pool(h):
            h1, h2 = _split(h)
            hh = jnp.concatenate([h1, h2], axis=0)

            def chunk4(j4, carry):
                m, denom, racc = carry
                for u in range(_NCHUNK):
                    j = u
                    tsl = pl.ds(j * _CHUNK, _CHUNK)
                    segc = batch_ref[:, tsl]
                    oh = iota_b == segc
                    sab = _mm(hh, x1t_ref[:, tsl])
                    s = sab[:_B] + sab[_B:] + _mm(h1, x2t_ref[:, tsl])
                    smask = jnp.where(oh, s, _NEG)
                    m_new = jnp.maximum(
                        m, jnp.max(smask, axis=1, keepdims=True))
                    scale = jnp.exp(m - m_new)
                    p = jnp.exp(smask - m_new)
                    denom = (denom * scale
                             + jnp.sum(p, axis=1, keepdims=True))
                    p1, p2 = _split(p)
                    pp = jnp.concatenate([p1, p2], axis=0)
                    rab = _mm(pp, x1r_ref[tsl, :])
                    racc = racc * scale + (rab[:_B] + rab[_B:])
                    m = m_new
                return m, denom, racc

            m0 = jnp.full((_B, 1), _NEG, _F32)
            z1 = jnp.zeros((_B, 1), _F32)
            z2 = jnp.zeros((_B, _C), _F32)
            _, denom, racc = chunk4(0, (m0, z1, z2))
            return racc / (denom + 1e-16)

        qs = qstar_ref[...]
        h, c = _lstm_act(gates_2(qs, h, True) + bias, c)
        r = pool(h)
        for _ in range(_STEPS - 1):
            h, c = _lstm_act(gates_2(h, r, False) + bias, c)
            r = pool(h)

        out_ref[...] = jnp.concatenate([h, r], axis=1)


@functools.partial(jax.jit, static_argnames=("interpret",))
def _run_fused(x, seg_row, q_star, w_rows, W_ih, W_hh, bias, eye,
               interpret=False):
    grid = (_NCHUNK + 1,)
    return pl.pallas_call(
        _fused_body,
        grid=grid,
        in_specs=[
            pl.BlockSpec((_CHUNK, _C),
                         lambda i: (jnp.minimum(i, _NCHUNK - 1), 0)),
            pl.BlockSpec((1, _NTOK), lambda i: (0, 0)),
            pl.BlockSpec((_B, 2 * _C), lambda i: (0, 0)),
            pl.BlockSpec((_B, _L), lambda i: (0, 0)),
            pl.BlockSpec((4 * _C, 2 * _C), lambda i: (0, 0)),
            pl.BlockSpec((4 * _C, _C), lambda i: (0, 0)),
            pl.BlockSpec((_B, 4 * _C), lambda i: (0, 0)),
            pl.BlockSpec((_C, _C), lambda i: (0, 0)),
        ],
        out_specs=pl.BlockSpec((_B, 2 * _C), lambda i: (0, 0)),
        out_shape=jax.ShapeDtypeStruct((_B, 2 * _C), _F32),
        scratch_shapes=[
            pltpu.VMEM((_NTOK, _C), _BF16),
            pltpu.VMEM((_NTOK, _C), _BF16),
            pltpu.VMEM((_C, _NTOK), _BF16),
            pltpu.VMEM((_C, _NTOK), _BF16),
            pltpu.VMEM((4 * _C, _C), _F32),
            pltpu.VMEM((_B, _C), _F32),
            pltpu.VMEM((_B, 4 * _C), _F32),
        ],
        interpret=interpret,
    )(x, seg_row, q_star, w_rows, W_ih, W_hh, bias, eye)


def kernel(x, batch, q_star, bank_s_list, bank_s, index, cuda,
           W_ih, W_hh, b_ih, b_hh, interpret=False):
    w_rows = jax.lax.dynamic_slice_in_dim(
        bank_s_list, index, 1, axis=1).reshape(_B, _L)
    seg_row = batch.astype(jnp.int32).reshape(1, _NTOK)
    bias = jnp.broadcast_to((b_ih + b_hh).reshape(1, 4 * _C), (_B, 4 * _C))
    eye = jnp.eye(_C, dtype=_BF16)
    return _run_fused(x, seg_row, q_star, w_rows, W_ih, W_hh, bias, eye,
                      interpret=interpret)


# speedup vs baseline: 3.0471x; 1.2509x over previous
import functools

import jax
import jax.numpy as jnp
from jax.experimental import pallas as pl
from jax.experimental.pallas import tpu as pltpu

_C = 512
_B = 8
_L = 1024
_NTOK = _B * _L
_STEPS = 3
_CHUNK = 1024
_NCHUNK = _NTOK // _CHUNK
_GCHUNK = 512
_PREC_MM = jax.lax.Precision.DEFAULT
_NEG = -1e30
_F32 = jnp.float32
_BF16 = jnp.bfloat16


def _lstm_act(gates, c):
    ig = jax.nn.sigmoid(gates[:, 0 * _C:1 * _C])
    fg = jax.nn.sigmoid(gates[:, 1 * _C:2 * _C])
    gg = jnp.tanh(gates[:, 2 * _C:3 * _C])
    og = jax.nn.sigmoid(gates[:, 3 * _C:4 * _C])
    c = fg * c + ig * gg
    return og * jnp.tanh(c), c


def _split(v):
    hi = v.astype(_BF16)
    lo = (v - hi.astype(_F32)).astype(_BF16)
    return hi, lo


def _mm(a, b):
    return jax.lax.dot_general(a, b, (((1,), (0,)), ((), ())),
                               precision=_PREC_MM,
                               preferred_element_type=_F32)


def _fused_body(xc_ref, batch_ref, qstar_ref, w_ref, wih_hbm, whh_hbm,
                b_ref, eye_ref, out_ref,
                x1r_ref, x1t_ref, x2t_ref, wih_ref, whh_ref, wc_ref,
                h_ref, g_ref, sem1, sem2):
    i = pl.program_id(0)

    @pl.when(i == 0)
    def _prefetch():
        pltpu.make_async_copy(wih_hbm, wih_ref, sem1).start()
        pltpu.make_async_copy(whh_hbm, whh_ref, sem2).start()

    @pl.when(i < _NCHUNK)
    def _build():
        xc = xc_ref[...]
        hi, lo = _split(xc)
        x1r_ref[pl.ds(i * _CHUNK, _CHUNK), :] = hi
        eye = eye_ref[...]
        th = jax.lax.dot_general(eye, hi, (((1,), (1,)), ((), ())),
                                 precision=_PREC_MM,
                                 preferred_element_type=_F32)
        x1t_ref[:, pl.ds(i * _CHUNK, _CHUNK)] = th.astype(_BF16)
        tl = jax.lax.dot_general(eye, lo, (((1,), (1,)), ((), ())),
                                 precision=_PREC_MM,
                                 preferred_element_type=_F32)
        x2t_ref[:, pl.ds(i * _CHUNK, _CHUNK)] = tl.astype(_BF16)
        wrow = w_ref[pl.ds(i, 1), :]
        h_ref[pl.ds(i, 1), :] = jax.lax.dot_general(
            wrow, xc, (((1,), (0,)), ((), ())), precision=_PREC_MM)

    @pl.when(i == _NCHUNK)
    def _main():
        pltpu.make_async_copy(wih_hbm, wih_ref, sem1).wait()
        pltpu.make_async_copy(whh_hbm, whh_ref, sem2).wait()

        for g in range((4 * _C) // _GCHUNK):
            wc_ref[pl.ds(g * _GCHUNK, _GCHUNK), :] = (
                wih_ref[pl.ds(g * _GCHUNK, _GCHUNK), 0:_C]
                + whh_ref[pl.ds(g * _GCHUNK, _GCHUNK), :])

        h = h_ref[...]
        c = jnp.zeros((_B, _C), _F32)
        bias = b_ref[...]
        iota_b = jax.lax.broadcasted_iota(jnp.int32, (_B, _CHUNK), 0)

        def gates_2(lhs1, lhs2, split_w1):
            for g in range((4 * _C) // _GCHUNK):
                gsl = pl.ds(g * _GCHUNK, _GCHUNK)
                if split_w1:
                    w1c = wih_ref[gsl, :]
                    w2c = whh_ref[gsl, :]
                else:
                    w1c = wc_ref[gsl, :]
                    w2c = wih_ref[gsl, _C:2 * _C]
                g_ref[:, gsl] = (
                    jax.lax.dot_general(lhs1, w1c, (((1,), (1,)), ((), ())),
                                        precision=_PREC_MM)
                    + jax.lax.dot_general(lhs2, w2c, (((1,), (1,)), ((), ())),
                                          precision=_PREC_MM))
            return g_ref[...]

        def pool(h):
            h1, h2 = _split(h)
            hh = jnp.concatenate([h1, h2], axis=0)
            m = jnp.full((_B, 1), _NEG, _F32)
            denom = jnp.zeros((_B, 1), _F32)
            racc = jnp.zeros((_B, _C), _F32)
            for j in range(_NCHUNK):
                tsl = pl.ds(j * _CHUNK, _CHUNK)
                segc = batch_ref[:, tsl]
                oh = iota_b == segc
                sab = _mm(hh, x1t_ref[:, tsl])
                s = sab[:_B] + sab[_B:] + _mm(h1, x2t_ref[:, tsl])
                smask = jnp.where(oh, s, _NEG)
                m_new = jnp.maximum(m, jnp.max(smask, axis=1, keepdims=True))
                scale = jnp.exp(m - m_new)
                p = jnp.exp(smask - m_new)
                denom = denom * scale + jnp.sum(p, axis=1, keepdims=True)
                p1, p2 = _split(p)
                pp = jnp.concatenate([p1, p2], axis=0)
                rab = _mm(pp, x1r_ref[tsl, :])
                racc = racc * scale + (rab[:_B] + rab[_B:])
                m = m_new
            return racc / (denom + 1e-16)

        qs = qstar_ref[...]
        h, c = _lstm_act(gates_2(qs, h, True) + bias, c)
        r = pool(h)
        for _ in range(_STEPS - 1):
            h, c = _lstm_act(gates_2(h, r, False) + bias, c)
            r = pool(h)

        out_ref[...] = jnp.concatenate([h, r], axis=1)


@functools.partial(jax.jit, static_argnames=("interpret",))
def _run_fused(x, seg_row, q_star, w_rows, W_ih, W_hh, bias, eye,
               interpret=False):
    grid = (_NCHUNK + 1,)
    return pl.pallas_call(
        _fused_body,
        grid=grid,
        in_specs=[
            pl.BlockSpec((_CHUNK, _C),
                         lambda i: (jnp.minimum(i, _NCHUNK - 1), 0)),
            pl.BlockSpec((1, _NTOK), lambda i: (0, 0)),
            pl.BlockSpec((_B, 2 * _C), lambda i: (0, 0)),
            pl.BlockSpec((_B, _L), lambda i: (0, 0)),
            pl.BlockSpec(memory_space=pltpu.MemorySpace.HBM),
            pl.BlockSpec(memory_space=pltpu.MemorySpace.HBM),
            pl.BlockSpec((_B, 4 * _C), lambda i: (0, 0)),
            pl.BlockSpec((_C, _C), lambda i: (0, 0)),
        ],
        out_specs=pl.BlockSpec((_B, 2 * _C), lambda i: (0, 0)),
        out_shape=jax.ShapeDtypeStruct((_B, 2 * _C), _F32),
        scratch_shapes=[
            pltpu.VMEM((_NTOK, _C), _BF16),
            pltpu.VMEM((_C, _NTOK), _BF16),
            pltpu.VMEM((_C, _NTOK), _BF16),
            pltpu.VMEM((4 * _C, 2 * _C), _F32),
            pltpu.VMEM((4 * _C, _C), _F32),
            pltpu.VMEM((4 * _C, _C), _F32),
            pltpu.VMEM((_B, _C), _F32),
            pltpu.VMEM((_B, 4 * _C), _F32),
            pltpu.SemaphoreType.DMA,
            pltpu.SemaphoreType.DMA,
        ],
        interpret=interpret,
    )(x, seg_row, q_star, w_rows, W_ih, W_hh, bias, eye)


def kernel(x, batch, q_star, bank_s_list, bank_s, index, cuda,
           W_ih, W_hh, b_ih, b_hh, interpret=False):
    w_rows = jax.lax.dynamic_slice_in_dim(
        bank_s_list, index, 1, axis=1).reshape(_B, _L)
    seg_row = batch.astype(jnp.int32).reshape(1, _NTOK)
    bias = jnp.broadcast_to((b_ih + b_hh).reshape(1, 4 * _C), (_B, 4 * _C))
    eye = jnp.eye(_C, dtype=_BF16)
    return _run_fused(x, seg_row, q_star, w_rows, W_ih, W_hh, bias, eye,
                      interpret=interpret)


# speedup vs baseline: 3.0528x; 1.0019x over previous
import functools

import jax
import jax.numpy as jnp
from jax.experimental import pallas as pl
from jax.experimental.pallas import tpu as pltpu

_C = 512
_B = 8
_L = 1024
_NTOK = _B * _L
_STEPS = 3
_CHUNK = 1024
_NCHUNK = _NTOK // _CHUNK
_GCHUNK = 512
_PREC_MM = jax.lax.Precision.DEFAULT
_NEG = -1e30
_F32 = jnp.float32
_BF16 = jnp.bfloat16


def _lstm_act(gates, c):
    ig = jax.nn.sigmoid(gates[:, 0 * _C:1 * _C])
    fg = jax.nn.sigmoid(gates[:, 1 * _C:2 * _C])
    gg = jnp.tanh(gates[:, 2 * _C:3 * _C])
    og = jax.nn.sigmoid(gates[:, 3 * _C:4 * _C])
    c = fg * c + ig * gg
    return og * jnp.tanh(c), c


def _split(v):
    hi = v.astype(_BF16)
    lo = (v - hi.astype(_F32)).astype(_BF16)
    return hi, lo


def _mm(a, b):
    return jax.lax.dot_general(a, b, (((1,), (0,)), ((), ())),
                               precision=_PREC_MM,
                               preferred_element_type=_F32)


def _fused_body(xc_ref, batch_ref, qstar_ref, w_ref, wih_hbm, whh_hbm,
                b_ref, eye_ref, out_ref,
                x1r_ref, x1t_ref, x2t_ref, wih_ref, whh_ref,
                wihb_ref, whhb_ref, wc_ref, h_ref, g_ref, sem1, sem2):
    i = pl.program_id(0)

    @pl.when(i == 0)
    def _prefetch():
        pltpu.make_async_copy(wih_hbm, wih_ref, sem1).start()
        pltpu.make_async_copy(whh_hbm, whh_ref, sem2).start()

    @pl.when(i < _NCHUNK)
    def _build():
        xc = xc_ref[...]
        hi, lo = _split(xc)
        x1r_ref[pl.ds(i * _CHUNK, _CHUNK), :] = hi
        eye = eye_ref[...]
        hilo = jnp.concatenate([hi, lo], axis=0)
        t2 = jax.lax.dot_general(eye, hilo, (((1,), (1,)), ((), ())),
                                 precision=_PREC_MM,
                                 preferred_element_type=_F32)
        t2 = t2.astype(_BF16)
        x1t_ref[:, pl.ds(i * _CHUNK, _CHUNK)] = t2[:, :_CHUNK]
        x2t_ref[:, pl.ds(i * _CHUNK, _CHUNK)] = t2[:, _CHUNK:]
        wrow = w_ref[pl.ds(i, 1), :]
        h_ref[pl.ds(i, 1), :] = jax.lax.dot_general(
            wrow, xc, (((1,), (0,)), ((), ())), precision=_PREC_MM)

    @pl.when(i == _NCHUNK)
    def _main():
        pltpu.make_async_copy(wih_hbm, wih_ref, sem1).wait()
        pltpu.make_async_copy(whh_hbm, whh_ref, sem2).wait()

        for g in range((4 * _C) // _GCHUNK):
            gsl = pl.ds(g * _GCHUNK, _GCHUNK)
            wihc = wih_ref[gsl, :]
            whhc = whh_ref[gsl, :]
            wihb_ref[gsl, :] = wihc.astype(_BF16)
            whhb_ref[gsl, :] = whhc.astype(_BF16)
            wc_ref[gsl, :] = (wihc[:, 0:_C] + whhc).astype(_BF16)

        h = h_ref[...]
        c = jnp.zeros((_B, _C), _F32)
        bias = b_ref[...]
        iota_b = jax.lax.broadcasted_iota(jnp.int32, (_B, _CHUNK), 0)

        def gates_2(lhs1, lhs2, split_w1):
            l1 = lhs1.astype(_BF16)
            l2 = lhs2.astype(_BF16)
            for g in range((4 * _C) // _GCHUNK):
                gsl = pl.ds(g * _GCHUNK, _GCHUNK)
                if split_w1:
                    w1c = wihb_ref[gsl, :]
                    w2c = whhb_ref[gsl, :]
                else:
                    w1c = wc_ref[gsl, :]
                    w2c = wihb_ref[gsl, _C:2 * _C]
                g_ref[:, gsl] = (
                    jax.lax.dot_general(l1, w1c, (((1,), (1,)), ((), ())),
                                        precision=_PREC_MM,
                                        preferred_element_type=_F32)
                    + jax.lax.dot_general(l2, w2c, (((1,), (1,)), ((), ())),
                                          precision=_PREC_MM,
                                          preferred_element_type=_F32))
            return g_ref[...]

        def pool(h):
            h1, h2 = _split(h)
            hh = jnp.concatenate([h1, h2], axis=0)
            m = jnp.full((_B, 1), _NEG, _F32)
            denom = jnp.zeros((_B, 1), _F32)
            racc = jnp.zeros((_B, _C), _F32)
            for j in range(_NCHUNK):
                tsl = pl.ds(j * _CHUNK, _CHUNK)
                segc = batch_ref[:, tsl]
                oh = iota_b == segc
                sab = _mm(hh, x1t_ref[:, tsl])
                s = sab[:_B] + sab[_B:] + _mm(h1, x2t_ref[:, tsl])
                smask = jnp.where(oh, s, _NEG)
                m_new = jnp.maximum(m, jnp.max(smask, axis=1, keepdims=True))
                scale = jnp.exp(m - m_new)
                p = jnp.exp(smask - m_new)
                denom = denom * scale + jnp.sum(p, axis=1, keepdims=True)
                p1, p2 = _split(p)
                pp = jnp.concatenate([p1, p2], axis=0)
                rab = _mm(pp, x1r_ref[tsl, :])
                racc = racc * scale + (rab[:_B] + rab[_B:])
                m = m_new
            return racc / (denom + 1e-16)

        qs = qstar_ref[...]
        h, c = _lstm_act(gates_2(qs, h, True) + bias, c)
        r = pool(h)
        for _ in range(_STEPS - 1):
            h, c = _lstm_act(gates_2(h, r, False) + bias, c)
            r = pool(h)

        out_ref[...] = jnp.concatenate([h, r], axis=1)


@functools.partial(jax.jit, static_argnames=("interpret",))
def _run_fused(x, seg_row, q_star, w_rows, W_ih, W_hh, bias, eye,
               interpret=False):
    grid = (_NCHUNK + 1,)
    return pl.pallas_call(
        _fused_body,
        grid=grid,
        in_specs=[
            pl.BlockSpec((_CHUNK, _C),
                         lambda i: (jnp.minimum(i, _NCHUNK - 1), 0)),
            pl.BlockSpec((1, _NTOK), lambda i: (0, 0)),
            pl.BlockSpec((_B, 2 * _C), lambda i: (0, 0)),
            pl.BlockSpec((_B, _L), lambda i: (0, 0)),
            pl.BlockSpec(memory_space=pltpu.MemorySpace.HBM),
            pl.BlockSpec(memory_space=pltpu.MemorySpace.HBM),
            pl.BlockSpec((_B, 4 * _C), lambda i: (0, 0)),
            pl.BlockSpec((_C, _C), lambda i: (0, 0)),
        ],
        out_specs=pl.BlockSpec((_B, 2 * _C), lambda i: (0, 0)),
        out_shape=jax.ShapeDtypeStruct((_B, 2 * _C), _F32),
        scratch_shapes=[
            pltpu.VMEM((_NTOK, _C), _BF16),
            pltpu.VMEM((_C, _NTOK), _BF16),
            pltpu.VMEM((_C, _NTOK), _BF16),
            pltpu.VMEM((4 * _C, 2 * _C), _F32),
            pltpu.VMEM((4 * _C, _C), _F32),
            pltpu.VMEM((4 * _C, 2 * _C), _BF16),
            pltpu.VMEM((4 * _C, _C), _BF16),
            pltpu.VMEM((4 * _C, _C), _BF16),
            pltpu.VMEM((_B, _C), _F32),
            pltpu.VMEM((_B, 4 * _C), _F32),
            pltpu.SemaphoreType.DMA,
            pltpu.SemaphoreType.DMA,
        ],
        interpret=interpret,
    )(x, seg_row, q_star, w_rows, W_ih, W_hh, bias, eye)


def kernel(x, batch, q_star, bank_s_list, bank_s, index, cuda,
           W_ih, W_hh, b_ih, b_hh, interpret=False):
    w_rows = jax.lax.dynamic_slice_in_dim(
        bank_s_list, index, 1, axis=1).reshape(_B, _L)
    seg_row = batch.astype(jnp.int32).reshape(1, _NTOK)
    bias = jnp.broadcast_to((b_ih + b_hh).reshape(1, 4 * _C), (_B, 4 * _C))
    eye = jnp.eye(_C, dtype=_BF16)
    return _run_fused(x, seg_row, q_star, w_rows, W_ih, W_hh, bias, eye,
                      interpret=interpret)
